# baseline (speedup 1.0000x reference)
import jax
import jax.numpy as jnp
from jax.experimental import pallas as pl
from jax.experimental.pallas import tpu as pltpu

_ACTION_DIM = 7
_BLOCK_ROWS = 2048


def _sanitize(x):
    return jnp.where(jnp.isfinite(x), x, 0.0)


def _mask_kernel(pos_ref, port_ref, out_ref, acc_ref):
    phase = pl.program_id(0)
    blk = pl.program_id(1)

    p0 = _sanitize(pos_ref[:, 0:1])
    has = p0 > 0.5

    @pl.when(phase == 0)
    def _reduce():
        block_min = jnp.min(jnp.where(has, 1, 0))
        prev = jnp.where(blk == 0, 1, acc_ref[0])
        acc_ref[0] = jnp.minimum(prev, block_min)

    @pl.when(phase == 1)
    def _emit():
        p4 = _sanitize(pos_ref[:, 4:5])
        ex = _sanitize(port_ref[:, 2:3])
        hx = ex >= 0.9
        asl = p4 >= 0.9
        all_has = acc_ref[0] == 1

        not_hx = jnp.logical_not(hx)
        buy = jnp.logical_not(has) & not_hx
        c6 = has & not_hx & jnp.logical_not(jnp.logical_and(all_has, asl))

        n = out_ref.shape[0]
        one = jnp.int8(1)
        out_ref[:, 0:1] = jnp.full((n, 1), one, dtype=jnp.int8)
        buy8 = buy.astype(jnp.int8)
        has8 = has.astype(jnp.int8)
        out_ref[:, 1:2] = buy8
        out_ref[:, 2:3] = buy8
        out_ref[:, 3:4] = buy8
        out_ref[:, 4:5] = has8
        out_ref[:, 5:6] = has8
        out_ref[:, 6:7] = c6.astype(jnp.int8)


@jax.jit
def kernel(position, portfolio):
    position = position.astype(jnp.float32)
    portfolio = portfolio.astype(jnp.float32)
    batch = position.shape[0]
    nb = batch // _BLOCK_ROWS
    raw = pl.pallas_call(
        _mask_kernel,
        grid=(2, nb),
        in_specs=[
            pl.BlockSpec((_BLOCK_ROWS, 5), lambda p, b: (b, 0)),
            pl.BlockSpec((_BLOCK_ROWS, 8), lambda p, b: (b, 0)),
        ],
        out_specs=pl.BlockSpec((_BLOCK_ROWS, _ACTION_DIM), lambda p, b: (b * p, 0)),
        out_shape=jax.ShapeDtypeStruct((batch, _ACTION_DIM), jnp.int8),
        scratch_shapes=[pltpu.SMEM((1,), jnp.int32)],
    )(position, portfolio)
    return raw.astype(jnp.bool_)


# speedup vs baseline: 7.6841x; 7.6841x over previous
import jax
import jax.numpy as jnp
from jax.experimental import pallas as pl

_ACTION_DIM = 7


def _sanitize(x):
    return jnp.where(jnp.isfinite(x), x, 0.0)


def _mask_kernel(cols_ref, out_ref):
    n = out_ref.shape[1]
    x = cols_ref[...]
    p0 = _sanitize(x[0:n])
    p4 = _sanitize(x[n:2 * n])
    ex = _sanitize(x[2 * n:3 * n])

    has = p0 > 0.5
    hx = ex >= 0.9
    asl = p4 >= 0.9

    all_has = jnp.min(jnp.where(has, 1, 0)) == 1

    not_hx = jnp.logical_not(hx)
    buy = jnp.logical_not(has) & not_hx
    c6 = has & not_hx & jnp.logical_not(jnp.logical_and(all_has, asl))

    buy8 = buy.astype(jnp.int8).reshape(1, n)
    has8 = has.astype(jnp.int8).reshape(1, n)
    c68 = c6.astype(jnp.int8).reshape(1, n)

    out_ref[0:1, :] = jnp.ones((1, n), dtype=jnp.int8)
    out_ref[1:2, :] = buy8
    out_ref[2:3, :] = buy8
    out_ref[3:4, :] = buy8
    out_ref[4:5, :] = has8
    out_ref[5:6, :] = has8
    out_ref[6:7, :] = c68


@jax.jit
def kernel(position, portfolio):
    position = position.astype(jnp.float32)
    portfolio = portfolio.astype(jnp.float32)
    batch = position.shape[0]
    cols = jnp.concatenate(
        [position[:, 0], position[:, 4], portfolio[:, 2]], axis=0
    )
    raw = pl.pallas_call(
        _mask_kernel,
        out_shape=jax.ShapeDtypeStruct((_ACTION_DIM, batch), jnp.int8),
    )(cols)
    return raw.T.astype(jnp.bool_)
